# baseline (speedup 1.0000x reference)
import functools

import jax
import jax.numpy as jnp
from jax import lax
from jax.experimental import pallas as pl
from jax.experimental.pallas import tpu as pltpu
from jax.experimental.pallas import tpu_sc as plsc

N = 10000
E = 320000
D = 128
NCLS = 64
EPS_BN = 1e-5

NC = 2
NS = 16
NW = NC * NS
K = 128
NCHUNK = 80
EPT = NCHUNK * K
EPAD = NW * EPT
RPT = 624
TAIL0 = RPT * NS
TAIL = N - TAIL0
ACC_ROWS = N + 8

BLK = 1000
GRID = N // BLK


def _sc_spmm_body(table, srcr, dstr, zrows, out, sidx, didx, rows, acc,
                  gsems):
    c = lax.axis_index("c")
    s = lax.axis_index("s")
    wid = s * NC + c
    r0 = s * RPT

    @pl.when(c == 0)
    def _():
        pltpu.sync_copy(table.at[pl.ds(r0, RPT)], acc.at[pl.ds(r0, RPT)])

        @pl.when(s == NS - 1)
        def _():
            pltpu.sync_copy(table.at[pl.ds(TAIL0, TAIL)],
                            acc.at[pl.ds(TAIL0, TAIL)])
            pltpu.sync_copy(zrows.at[pl.ds(0, ACC_ROWS - N)],
                            acc.at[pl.ds(N, ACC_ROWS - N)])

    @pl.when(c == 1)
    def _():
        pltpu.sync_copy(zrows.at[pl.ds(0, RPT)], acc.at[pl.ds(r0, RPT)])

        @pl.when(s == NS - 1)
        def _():
            pltpu.sync_copy(zrows.at[pl.ds(0, TAIL + ACC_ROWS - N)],
                            acc.at[pl.ds(TAIL0, TAIL + ACC_ROWS - N)])

    plsc.subcore_barrier()

    pltpu.sync_copy(srcr.at[wid], sidx)
    pltpu.sync_copy(dstr.at[wid], didx)

    def chunk(i, carry):
        pltpu.async_copy(table.at[sidx.at[i]], rows, gsems).wait()
        pltpu.sync_copy(rows, acc.at[didx.at[i]], add=True)
        return carry

    lax.fori_loop(0, NCHUNK, chunk, 0)

    plsc.subcore_barrier()
    pltpu.sync_copy(acc.at[pl.ds(r0, RPT)], out.at[c].at[pl.ds(r0, RPT)])

    @pl.when(s == NS - 1)
    def _():
        pltpu.sync_copy(acc.at[pl.ds(TAIL0, TAIL)],
                        out.at[c].at[pl.ds(TAIL0, TAIL)])


@functools.cache
def _sc_spmm_kernel():
    return pl.kernel(
        _sc_spmm_body,
        out_type=jax.ShapeDtypeStruct((NC, N, D), jnp.float32),
        mesh=plsc.VectorSubcoreMesh(
            core_axis_name="c", subcore_axis_name="s",
            num_cores=NC, num_subcores=NS),
        scratch_types=[
            pltpu.VMEM((NCHUNK, K), jnp.int32),
            pltpu.VMEM((NCHUNK, K), jnp.int32),
            pltpu.VMEM((K, D), jnp.float32),
            pltpu.VMEM_SHARED((ACC_ROWS, D), jnp.float32),
            pltpu.SemaphoreType.DMA,
        ],
    )


def _sc_spmm(h, src_p, dst_p, zrows):
    return _sc_spmm_kernel()(h, src_p, dst_p, zrows)


def _linear_stats_body(p0, p1, w, b, z, stats, s1, s2):
    i = pl.program_id(0)
    agg = p0[...] + p1[...]
    zz = lax.dot_general(agg, w[...], (((1,), (1,)), ((), ())),
                         preferred_element_type=jnp.float32) + b[...]
    z[...] = zz

    @pl.when(i == 0)
    def _():
        s1[...] = jnp.zeros_like(s1)
        s2[...] = jnp.zeros_like(s2)

    s1[...] += jnp.sum(zz, axis=0, keepdims=True)
    s2[...] += jnp.sum(zz * zz, axis=0, keepdims=True)

    @pl.when(i == pl.num_programs(0) - 1)
    def _():
        mean = s1[...] / N
        var = s2[...] / N - mean * mean
        stats[...] = jnp.concatenate(
            [mean, lax.rsqrt(var + EPS_BN)], axis=0)


def _linear_stats(p0, p1, w, b):
    return pl.pallas_call(
        _linear_stats_body,
        grid=(GRID,),
        in_specs=[
            pl.BlockSpec((BLK, D), lambda i: (i, 0)),
            pl.BlockSpec((BLK, D), lambda i: (i, 0)),
            pl.BlockSpec((D, D), lambda i: (0, 0)),
            pl.BlockSpec((1, D), lambda i: (0, 0)),
        ],
        out_specs=[
            pl.BlockSpec((BLK, D), lambda i: (i, 0)),
            pl.BlockSpec((2, D), lambda i: (0, 0)),
        ],
        out_shape=[
            jax.ShapeDtypeStruct((N, D), jnp.float32),
            jax.ShapeDtypeStruct((2, D), jnp.float32),
        ],
        scratch_shapes=[
            pltpu.VMEM((1, D), jnp.float32),
            pltpu.VMEM((1, D), jnp.float32),
        ],
    )(p0, p1, w, b)


def _bn_relu_body(z, stats, h):
    mean = stats[0:1, :]
    scale = stats[1:2, :]
    h[...] = jnp.maximum((z[...] - mean) * scale, 0.0)


def _bn_relu(z, stats):
    return pl.pallas_call(
        _bn_relu_body,
        grid=(GRID,),
        in_specs=[
            pl.BlockSpec((BLK, D), lambda i: (i, 0)),
            pl.BlockSpec((2, D), lambda i: (0, 0)),
        ],
        out_specs=pl.BlockSpec((BLK, D), lambda i: (i, 0)),
        out_shape=jax.ShapeDtypeStruct((N, D), jnp.float32),
    )(z, stats)


def _final_body(z, stats, w, b, out):
    mean = stats[0:1, :]
    scale = stats[1:2, :]
    h = jnp.maximum((z[...] - mean) * scale, 0.0)
    out[...] = lax.dot_general(h, w[...], (((1,), (1,)), ((), ())),
                               preferred_element_type=jnp.float32) + b[...]


def _final(z, stats, w, b):
    return pl.pallas_call(
        _final_body,
        grid=(GRID,),
        in_specs=[
            pl.BlockSpec((BLK, D), lambda i: (i, 0)),
            pl.BlockSpec((2, D), lambda i: (0, 0)),
            pl.BlockSpec((NCLS, D), lambda i: (0, 0)),
            pl.BlockSpec((1, NCLS), lambda i: (0, 0)),
        ],
        out_specs=pl.BlockSpec((BLK, NCLS), lambda i: (i, 0)),
        out_shape=jax.ShapeDtypeStruct((N, NCLS), jnp.float32),
    )(z, stats, w, b)


def kernel(x, edge_index, W0, b0, W1, b1, W2, b2, W_last, b_last):
    dst = edge_index[0].astype(jnp.int32)
    src = edge_index[1].astype(jnp.int32)
    pad = EPAD - E
    src_p = jnp.concatenate(
        [src, jnp.zeros((pad,), jnp.int32)]).reshape(NW, NCHUNK, K)
    pad_dst = N + (jnp.arange(pad, dtype=jnp.int32) % (ACC_ROWS - N))
    dst_p = jnp.concatenate([dst, pad_dst]).reshape(NW, NCHUNK, K)
    zrows = jnp.zeros((RPT, D), jnp.float32)

    h = x
    z = stats = None
    for li, (w, b) in enumerate(((W0, b0), (W1, b1), (W2, b2))):
        parts = _sc_spmm(h, src_p, dst_p, zrows)
        z, stats = _linear_stats(parts[0], parts[1], w, b.reshape(1, D))
        if li < 2:
            h = _bn_relu(z, stats)
    return _final(z, stats, W_last, b_last.reshape(1, NCLS))


# speedup vs baseline: 1.7142x; 1.7142x over previous
import functools

import jax
import jax.numpy as jnp
from jax import lax
from jax.experimental import pallas as pl
from jax.experimental.pallas import tpu as pltpu
from jax.experimental.pallas import tpu_sc as plsc

N = 10000
E = 320000
D = 128
NCLS = 64
EPS_BN = 1e-5

NC = 2
NS = 16
NW = NC * NS
K = 128
NCHUNK = 79
EPT = NCHUNK * K
EPAD = NW * EPT
RPT = 624
TAIL0 = RPT * NS
TAIL = N - TAIL0
ACC_ROWS = N + 8

BLK = 1000
GRID = N // BLK


def _sc_spmm_body(table, srcr, dstr, zrows, out, sidx, didx0, didx1,
                  rows0, rows1, acc, g0, g1, d0, d1):
    c = lax.axis_index("c")
    s = lax.axis_index("s")
    wid = s * NC + c
    r0 = s * RPT

    @pl.when(c == 0)
    def _():
        pltpu.sync_copy(table.at[pl.ds(r0, RPT)], acc.at[pl.ds(r0, RPT)])

        @pl.when(s == NS - 1)
        def _():
            pltpu.sync_copy(table.at[pl.ds(TAIL0, TAIL)],
                            acc.at[pl.ds(TAIL0, TAIL)])
            pltpu.sync_copy(zrows.at[pl.ds(0, ACC_ROWS - N)],
                            acc.at[pl.ds(N, ACC_ROWS - N)])

    @pl.when(c == 1)
    def _():
        pltpu.sync_copy(zrows.at[pl.ds(0, RPT)], acc.at[pl.ds(r0, RPT)])

        @pl.when(s == NS - 1)
        def _():
            pltpu.sync_copy(zrows.at[pl.ds(0, TAIL + ACC_ROWS - N)],
                            acc.at[pl.ds(TAIL0, TAIL + ACC_ROWS - N)])

    plsc.subcore_barrier()

    pltpu.sync_copy(srcr.at[wid], sidx)

    def step(i, rows_c, g_c, didx_c, d_c, rows_n, g_n, didx_n, d_n):
        pltpu.make_async_copy(table.at[sidx.at[i]], rows_c, g_c).wait()

        @pl.when(i + 1 < NCHUNK)
        def _():
            pltpu.async_copy(dstr.at[wid].at[pl.ds(i + 1, 1)], didx_n, d_n)
            pltpu.async_copy(table.at[sidx.at[i + 1]], rows_n, g_n)

        pltpu.make_async_copy(
            dstr.at[wid].at[pl.ds(i, 1)], didx_c, d_c).wait()
        pltpu.sync_copy(rows_c, acc.at[didx_c.at[0]], add=True)

    pltpu.async_copy(dstr.at[wid].at[pl.ds(0, 1)], didx0, d0)
    pltpu.async_copy(table.at[sidx.at[0]], rows0, g0)

    def chunk(i, carry):
        @pl.when(i % 2 == 0)
        def _():
            step(i, rows0, g0, didx0, d0, rows1, g1, didx1, d1)

        @pl.when(i % 2 == 1)
        def _():
            step(i, rows1, g1, didx1, d1, rows0, g0, didx0, d0)

        return carry

    lax.fori_loop(0, NCHUNK, chunk, 0)

    plsc.subcore_barrier()
    pltpu.sync_copy(acc.at[pl.ds(r0, RPT)], out.at[c].at[pl.ds(r0, RPT)])

    @pl.when(s == NS - 1)
    def _():
        pltpu.sync_copy(acc.at[pl.ds(TAIL0, TAIL)],
                        out.at[c].at[pl.ds(TAIL0, TAIL)])


@functools.cache
def _sc_spmm_kernel():
    return pl.kernel(
        _sc_spmm_body,
        out_type=jax.ShapeDtypeStruct((NC, N, D), jnp.float32),
        mesh=plsc.VectorSubcoreMesh(
            core_axis_name="c", subcore_axis_name="s",
            num_cores=NC, num_subcores=NS),
        scratch_types=[
            pltpu.VMEM((NCHUNK, K), jnp.int32),
            pltpu.VMEM((1, K), jnp.int32),
            pltpu.VMEM((1, K), jnp.int32),
            pltpu.VMEM((K, D), jnp.float32),
            pltpu.VMEM((K, D), jnp.float32),
            pltpu.VMEM_SHARED((ACC_ROWS, D), jnp.float32),
            pltpu.SemaphoreType.DMA,
            pltpu.SemaphoreType.DMA,
            pltpu.SemaphoreType.DMA,
            pltpu.SemaphoreType.DMA,
        ],
    )


def _sc_spmm(h, src_p, dst_p, zrows):
    return _sc_spmm_kernel()(h, src_p, dst_p, zrows)


def _linear_stats_body(p0, p1, w, b, z, stats, s1, s2):
    i = pl.program_id(0)
    agg = p0[...] + p1[...]
    zz = lax.dot_general(agg, w[...], (((1,), (1,)), ((), ())),
                         preferred_element_type=jnp.float32) + b[...]
    z[...] = zz

    @pl.when(i == 0)
    def _():
        s1[...] = jnp.zeros_like(s1)
        s2[...] = jnp.zeros_like(s2)

    s1[...] += jnp.sum(zz, axis=0, keepdims=True)
    s2[...] += jnp.sum(zz * zz, axis=0, keepdims=True)

    @pl.when(i == pl.num_programs(0) - 1)
    def _():
        mean = s1[...] / N
        var = s2[...] / N - mean * mean
        stats[...] = jnp.concatenate(
            [mean, lax.rsqrt(var + EPS_BN)], axis=0)


def _linear_stats(p0, p1, w, b):
    return pl.pallas_call(
        _linear_stats_body,
        grid=(GRID,),
        in_specs=[
            pl.BlockSpec((BLK, D), lambda i: (i, 0)),
            pl.BlockSpec((BLK, D), lambda i: (i, 0)),
            pl.BlockSpec((D, D), lambda i: (0, 0)),
            pl.BlockSpec((1, D), lambda i: (0, 0)),
        ],
        out_specs=[
            pl.BlockSpec((BLK, D), lambda i: (i, 0)),
            pl.BlockSpec((2, D), lambda i: (0, 0)),
        ],
        out_shape=[
            jax.ShapeDtypeStruct((N, D), jnp.float32),
            jax.ShapeDtypeStruct((2, D), jnp.float32),
        ],
        scratch_shapes=[
            pltpu.VMEM((1, D), jnp.float32),
            pltpu.VMEM((1, D), jnp.float32),
        ],
    )(p0, p1, w, b)


def _bn_relu_body(z, stats, h):
    mean = stats[0:1, :]
    scale = stats[1:2, :]
    h[...] = jnp.maximum((z[...] - mean) * scale, 0.0)


def _bn_relu(z, stats):
    return pl.pallas_call(
        _bn_relu_body,
        grid=(GRID,),
        in_specs=[
            pl.BlockSpec((BLK, D), lambda i: (i, 0)),
            pl.BlockSpec((2, D), lambda i: (0, 0)),
        ],
        out_specs=pl.BlockSpec((BLK, D), lambda i: (i, 0)),
        out_shape=jax.ShapeDtypeStruct((N, D), jnp.float32),
    )(z, stats)


def _final_body(z, stats, w, b, out):
    mean = stats[0:1, :]
    scale = stats[1:2, :]
    h = jnp.maximum((z[...] - mean) * scale, 0.0)
    out[...] = lax.dot_general(h, w[...], (((1,), (1,)), ((), ())),
                               preferred_element_type=jnp.float32) + b[...]


def _final(z, stats, w, b):
    return pl.pallas_call(
        _final_body,
        grid=(GRID,),
        in_specs=[
            pl.BlockSpec((BLK, D), lambda i: (i, 0)),
            pl.BlockSpec((2, D), lambda i: (0, 0)),
            pl.BlockSpec((NCLS, D), lambda i: (0, 0)),
            pl.BlockSpec((1, NCLS), lambda i: (0, 0)),
        ],
        out_specs=pl.BlockSpec((BLK, NCLS), lambda i: (i, 0)),
        out_shape=jax.ShapeDtypeStruct((N, NCLS), jnp.float32),
    )(z, stats, w, b)


def kernel(x, edge_index, W0, b0, W1, b1, W2, b2, W_last, b_last):
    dst = edge_index[0].astype(jnp.int32)
    src = edge_index[1].astype(jnp.int32)
    pad = EPAD - E
    src_p = jnp.concatenate(
        [src, jnp.zeros((pad,), jnp.int32)]).reshape(NW, NCHUNK, K)
    pad_dst = N + (jnp.arange(pad, dtype=jnp.int32) % (ACC_ROWS - N))
    dst_p = jnp.concatenate([dst, pad_dst]).reshape(NW, NCHUNK, K)
    zrows = jnp.zeros((RPT, D), jnp.float32)

    h = x
    z = stats = None
    for li, (w, b) in enumerate(((W0, b0), (W1, b1), (W2, b2))):
        parts = _sc_spmm(h, src_p, dst_p, zrows)
        z, stats = _linear_stats(parts[0], parts[1], w, b.reshape(1, D))
        if li < 2:
            h = _bn_relu(z, stats)
    return _final(z, stats, W_last, b_last.reshape(1, NCLS))


# speedup vs baseline: 1.9371x; 1.1301x over previous
import functools

import jax
import jax.numpy as jnp
from jax import lax
from jax.experimental import pallas as pl
from jax.experimental.pallas import tpu as pltpu
from jax.experimental.pallas import tpu_sc as plsc

N = 10000
E = 320000
D = 128
NCLS = 64
EPS_BN = 1e-5

NC = 2
NS = 16
NW = NC * NS
K = 128
NCHUNK = 79
EPT = NCHUNK * K
EPAD = NW * EPT
RPT = 624
TAIL0 = RPT * NS
TAIL = N - TAIL0
PAD_PER = EPT - E // NW
ACC_ROWS = N + PAD_PER

BLK = 1000
GRID = N // BLK


def _sc_spmm_body(table, srcr, dstr, zrows, out, sidx, didx0, didx1,
                  rows0, rows1, acc, g0, g1, d0, d1):
    c = lax.axis_index("c")
    s = lax.axis_index("s")
    wid = s * NC + c
    r0 = s * RPT

    @pl.when(c == 0)
    def _():
        pltpu.sync_copy(table.at[pl.ds(r0, RPT)], acc.at[pl.ds(r0, RPT)])

        @pl.when(s == NS - 1)
        def _():
            pltpu.sync_copy(table.at[pl.ds(TAIL0, TAIL)],
                            acc.at[pl.ds(TAIL0, TAIL)])
            pltpu.sync_copy(zrows.at[pl.ds(0, ACC_ROWS - N)],
                            acc.at[pl.ds(N, ACC_ROWS - N)])

    @pl.when(c == 1)
    def _():
        pltpu.sync_copy(zrows.at[pl.ds(0, RPT)], acc.at[pl.ds(r0, RPT)])

        @pl.when(s == NS - 1)
        def _():
            pltpu.sync_copy(zrows.at[pl.ds(0, TAIL + ACC_ROWS - N)],
                            acc.at[pl.ds(TAIL0, TAIL + ACC_ROWS - N)])

    plsc.subcore_barrier()

    pltpu.sync_copy(srcr.at[wid], sidx)

    def step(i, rows_c, g_c, didx_c, d_c, rows_n, g_n, didx_n, d_n):
        pltpu.make_async_copy(table.at[sidx.at[i]], rows_c, g_c).wait()

        @pl.when(i + 1 < NCHUNK)
        def _():
            pltpu.async_copy(dstr.at[wid].at[pl.ds(i + 1, 1)], didx_n, d_n)
            pltpu.async_copy(table.at[sidx.at[i + 1]], rows_n, g_n)

        pltpu.make_async_copy(
            dstr.at[wid].at[pl.ds(i, 1)], didx_c, d_c).wait()
        pltpu.sync_copy(rows_c, acc.at[didx_c.at[0]], add=True)

    pltpu.async_copy(dstr.at[wid].at[pl.ds(0, 1)], didx0, d0)
    pltpu.async_copy(table.at[sidx.at[0]], rows0, g0)

    def chunk(i, carry):
        @pl.when(i % 2 == 0)
        def _():
            step(i, rows0, g0, didx0, d0, rows1, g1, didx1, d1)

        @pl.when(i % 2 == 1)
        def _():
            step(i, rows1, g1, didx1, d1, rows0, g0, didx0, d0)

        return carry

    lax.fori_loop(0, NCHUNK, chunk, 0)

    plsc.subcore_barrier()
    pltpu.sync_copy(acc.at[pl.ds(r0, RPT)], out.at[c].at[pl.ds(r0, RPT)])

    @pl.when(s == NS - 1)
    def _():
        pltpu.sync_copy(acc.at[pl.ds(TAIL0, TAIL)],
                        out.at[c].at[pl.ds(TAIL0, TAIL)])


@functools.cache
def _sc_spmm_kernel():
    return pl.kernel(
        _sc_spmm_body,
        out_type=jax.ShapeDtypeStruct((NC, N, D), jnp.float32),
        mesh=plsc.VectorSubcoreMesh(
            core_axis_name="c", subcore_axis_name="s",
            num_cores=NC, num_subcores=NS),
        scratch_types=[
            pltpu.VMEM((NCHUNK, K), jnp.int32),
            pltpu.VMEM((1, K), jnp.int32),
            pltpu.VMEM((1, K), jnp.int32),
            pltpu.VMEM((K, D), jnp.float32),
            pltpu.VMEM((K, D), jnp.float32),
            pltpu.VMEM_SHARED((ACC_ROWS, D), jnp.float32),
            pltpu.SemaphoreType.DMA,
            pltpu.SemaphoreType.DMA,
            pltpu.SemaphoreType.DMA,
            pltpu.SemaphoreType.DMA,
        ],
    )


def _sc_spmm(h, src_p, dst_p, zrows):
    return _sc_spmm_kernel()(h, src_p, dst_p, zrows)


def _linear_stats_body(p0, p1, w, b, z, stats, s1, s2):
    i = pl.program_id(0)
    agg = p0[...] + p1[...]
    zz = lax.dot_general(agg, w[...], (((1,), (1,)), ((), ())),
                         preferred_element_type=jnp.float32) + b[...]
    z[...] = zz

    @pl.when(i == 0)
    def _():
        s1[...] = jnp.zeros_like(s1)
        s2[...] = jnp.zeros_like(s2)

    s1[...] += jnp.sum(zz, axis=0, keepdims=True)
    s2[...] += jnp.sum(zz * zz, axis=0, keepdims=True)

    @pl.when(i == pl.num_programs(0) - 1)
    def _():
        mean = s1[...] / N
        var = s2[...] / N - mean * mean
        stats[...] = jnp.concatenate(
            [mean, lax.rsqrt(var + EPS_BN)], axis=0)


def _linear_stats(p0, p1, w, b):
    return pl.pallas_call(
        _linear_stats_body,
        grid=(GRID,),
        in_specs=[
            pl.BlockSpec((BLK, D), lambda i: (i, 0)),
            pl.BlockSpec((BLK, D), lambda i: (i, 0)),
            pl.BlockSpec((D, D), lambda i: (0, 0)),
            pl.BlockSpec((1, D), lambda i: (0, 0)),
        ],
        out_specs=[
            pl.BlockSpec((BLK, D), lambda i: (i, 0)),
            pl.BlockSpec((2, D), lambda i: (0, 0)),
        ],
        out_shape=[
            jax.ShapeDtypeStruct((N, D), jnp.float32),
            jax.ShapeDtypeStruct((2, D), jnp.float32),
        ],
        scratch_shapes=[
            pltpu.VMEM((1, D), jnp.float32),
            pltpu.VMEM((1, D), jnp.float32),
        ],
    )(p0, p1, w, b)


def _bn_relu_body(z, stats, h):
    mean = stats[0:1, :]
    scale = stats[1:2, :]
    h[...] = jnp.maximum((z[...] - mean) * scale, 0.0)


def _bn_relu(z, stats):
    return pl.pallas_call(
        _bn_relu_body,
        grid=(GRID,),
        in_specs=[
            pl.BlockSpec((BLK, D), lambda i: (i, 0)),
            pl.BlockSpec((2, D), lambda i: (0, 0)),
        ],
        out_specs=pl.BlockSpec((BLK, D), lambda i: (i, 0)),
        out_shape=jax.ShapeDtypeStruct((N, D), jnp.float32),
    )(z, stats)


def _final_body(z, stats, w, b, out):
    mean = stats[0:1, :]
    scale = stats[1:2, :]
    h = jnp.maximum((z[...] - mean) * scale, 0.0)
    out[...] = lax.dot_general(h, w[...], (((1,), (1,)), ((), ())),
                               preferred_element_type=jnp.float32) + b[...]


def _final(z, stats, w, b):
    return pl.pallas_call(
        _final_body,
        grid=(GRID,),
        in_specs=[
            pl.BlockSpec((BLK, D), lambda i: (i, 0)),
            pl.BlockSpec((2, D), lambda i: (0, 0)),
            pl.BlockSpec((NCLS, D), lambda i: (0, 0)),
            pl.BlockSpec((1, NCLS), lambda i: (0, 0)),
        ],
        out_specs=pl.BlockSpec((BLK, NCLS), lambda i: (i, 0)),
        out_shape=jax.ShapeDtypeStruct((N, NCLS), jnp.float32),
    )(z, stats, w, b)


def kernel(x, edge_index, W0, b0, W1, b1, W2, b2, W_last, b_last):
    dst = edge_index[0].astype(jnp.int32)
    src = edge_index[1].astype(jnp.int32)
    src2 = src.reshape(NW, E // NW)
    dst2 = dst.reshape(NW, E // NW)
    pad_src = jnp.zeros((NW, PAD_PER), jnp.int32)
    pad_dst = jnp.broadcast_to(
        N + jnp.arange(PAD_PER, dtype=jnp.int32), (NW, PAD_PER))
    src_p = jnp.concatenate([src2, pad_src], axis=1).reshape(NW, NCHUNK, K)
    dst_p = jnp.concatenate([dst2, pad_dst], axis=1).reshape(NW, NCHUNK, K)
    zrows = jnp.zeros((RPT, D), jnp.float32)

    h = x
    z = stats = None
    for li, (w, b) in enumerate(((W0, b0), (W1, b1), (W2, b2))):
        parts = _sc_spmm(h, src_p, dst_p, zrows)
        z, stats = _linear_stats(parts[0], parts[1], w, b.reshape(1, D))
        if li < 2:
            h = _bn_relu(z, stats)
    return _final(z, stats, W_last, b_last.reshape(1, NCLS))


# speedup vs baseline: 3.4246x; 1.7679x over previous
import functools

import jax
import jax.numpy as jnp
from jax import lax
from jax.experimental import pallas as pl
from jax.experimental.pallas import tpu as pltpu
from jax.experimental.pallas import tpu_sc as plsc

N = 10000
E = 320000
D = 128
NCLS = 64
EPS_BN = 1e-5

NC = 2
NS = 16
NW = NC * NS
K = 128
EPN = E // NW
NCHUNK = 78
TE = EPN - NCHUNK * K
RPT = 624
TAIL0 = RPT * NS
TAIL = N - TAIL0
ACC_ROWS = N

BLK = 1000
GRID = N // BLK


def _sc_spmm_body(table, srcr, dstr, srct, dstt, zrows, out,
                  sidx, didx0, didx1, rows0, rows1, stail, dtail, rtail,
                  acc, g0, g1, d0, d1):
    c = lax.axis_index("c")
    s = lax.axis_index("s")
    wid = s * NC + c
    r0 = s * RPT

    @pl.when(c == 0)
    def _():
        pltpu.sync_copy(table.at[pl.ds(r0, RPT)], acc.at[pl.ds(r0, RPT)])

        @pl.when(s == NS - 1)
        def _():
            pltpu.sync_copy(table.at[pl.ds(TAIL0, TAIL)],
                            acc.at[pl.ds(TAIL0, TAIL)])

    @pl.when(c == 1)
    def _():
        pltpu.sync_copy(zrows.at[pl.ds(0, RPT)], acc.at[pl.ds(r0, RPT)])

        @pl.when(s == NS - 1)
        def _():
            pltpu.sync_copy(zrows.at[pl.ds(0, TAIL)],
                            acc.at[pl.ds(TAIL0, TAIL)])

    plsc.subcore_barrier()

    pltpu.sync_copy(srct.at[wid], stail)
    pltpu.sync_copy(dstt.at[wid], dtail)
    pltpu.async_copy(table.at[stail], rtail, g0).wait()
    pltpu.sync_copy(rtail, acc.at[dtail], add=True)

    pltpu.sync_copy(srcr.at[wid], sidx)

    def step(i, rows_c, g_c, didx_c, d_c, rows_n, g_n, didx_n, d_n):
        pltpu.make_async_copy(table.at[sidx.at[i]], rows_c, g_c).wait()

        @pl.when(i + 1 < NCHUNK)
        def _():
            pltpu.async_copy(dstr.at[wid].at[pl.ds(i + 1, 1)], didx_n, d_n)
            pltpu.async_copy(table.at[sidx.at[i + 1]], rows_n, g_n)

        pltpu.make_async_copy(
            dstr.at[wid].at[pl.ds(i, 1)], didx_c, d_c).wait()
        pltpu.sync_copy(rows_c, acc.at[didx_c.at[0]], add=True)

    pltpu.async_copy(dstr.at[wid].at[pl.ds(0, 1)], didx0, d0)
    pltpu.async_copy(table.at[sidx.at[0]], rows0, g0)

    def chunk(i, carry):
        @pl.when(i % 2 == 0)
        def _():
            step(i, rows0, g0, didx0, d0, rows1, g1, didx1, d1)

        @pl.when(i % 2 == 1)
        def _():
            step(i, rows1, g1, didx1, d1, rows0, g0, didx0, d0)

        return carry

    lax.fori_loop(0, NCHUNK, chunk, 0)

    plsc.subcore_barrier()
    pltpu.sync_copy(acc.at[pl.ds(r0, RPT)], out.at[c].at[pl.ds(r0, RPT)])

    @pl.when(s == NS - 1)
    def _():
        pltpu.sync_copy(acc.at[pl.ds(TAIL0, TAIL)],
                        out.at[c].at[pl.ds(TAIL0, TAIL)])


@functools.cache
def _sc_spmm_kernel():
    return pl.kernel(
        _sc_spmm_body,
        out_type=jax.ShapeDtypeStruct((NC, N, D), jnp.float32),
        mesh=plsc.VectorSubcoreMesh(
            core_axis_name="c", subcore_axis_name="s",
            num_cores=NC, num_subcores=NS),
        scratch_types=[
            pltpu.VMEM((NCHUNK, K), jnp.int32),
            pltpu.VMEM((1, K), jnp.int32),
            pltpu.VMEM((1, K), jnp.int32),
            pltpu.VMEM((K, D), jnp.float32),
            pltpu.VMEM((K, D), jnp.float32),
            pltpu.VMEM((TE,), jnp.int32),
            pltpu.VMEM((TE,), jnp.int32),
            pltpu.VMEM((TE, D), jnp.float32),
            pltpu.VMEM_SHARED((ACC_ROWS, D), jnp.float32),
            pltpu.SemaphoreType.DMA,
            pltpu.SemaphoreType.DMA,
            pltpu.SemaphoreType.DMA,
            pltpu.SemaphoreType.DMA,
        ],
    )


def _sc_spmm(h, src_m, dst_m, src_t, dst_t, zrows):
    return _sc_spmm_kernel()(h, src_m, dst_m, src_t, dst_t, zrows)


def _linear_stats_body(p0, p1, w, b, z, stats, s1, s2):
    i = pl.program_id(0)
    agg = p0[...] + p1[...]
    zz = lax.dot_general(agg, w[...], (((1,), (1,)), ((), ())),
                         preferred_element_type=jnp.float32) + b[...]
    z[...] = zz

    @pl.when(i == 0)
    def _():
        s1[...] = jnp.zeros_like(s1)
        s2[...] = jnp.zeros_like(s2)

    s1[...] += jnp.sum(zz, axis=0, keepdims=True)
    s2[...] += jnp.sum(zz * zz, axis=0, keepdims=True)

    @pl.when(i == pl.num_programs(0) - 1)
    def _():
        mean = s1[...] / N
        var = s2[...] / N - mean * mean
        stats[...] = jnp.concatenate(
            [mean, lax.rsqrt(var + EPS_BN)], axis=0)


def _linear_stats(p0, p1, w, b):
    return pl.pallas_call(
        _linear_stats_body,
        grid=(GRID,),
        in_specs=[
            pl.BlockSpec((BLK, D), lambda i: (i, 0)),
            pl.BlockSpec((BLK, D), lambda i: (i, 0)),
            pl.BlockSpec((D, D), lambda i: (0, 0)),
            pl.BlockSpec((1, D), lambda i: (0, 0)),
        ],
        out_specs=[
            pl.BlockSpec((BLK, D), lambda i: (i, 0)),
            pl.BlockSpec((2, D), lambda i: (0, 0)),
        ],
        out_shape=[
            jax.ShapeDtypeStruct((N, D), jnp.float32),
            jax.ShapeDtypeStruct((2, D), jnp.float32),
        ],
        scratch_shapes=[
            pltpu.VMEM((1, D), jnp.float32),
            pltpu.VMEM((1, D), jnp.float32),
        ],
    )(p0, p1, w, b)


def _bn_relu_body(z, stats, h):
    mean = stats[0:1, :]
    scale = stats[1:2, :]
    h[...] = jnp.maximum((z[...] - mean) * scale, 0.0)


def _bn_relu(z, stats):
    return pl.pallas_call(
        _bn_relu_body,
        grid=(GRID,),
        in_specs=[
            pl.BlockSpec((BLK, D), lambda i: (i, 0)),
            pl.BlockSpec((2, D), lambda i: (0, 0)),
        ],
        out_specs=pl.BlockSpec((BLK, D), lambda i: (i, 0)),
        out_shape=jax.ShapeDtypeStruct((N, D), jnp.float32),
    )(z, stats)


def _final_body(z, stats, w, b, out):
    mean = stats[0:1, :]
    scale = stats[1:2, :]
    h = jnp.maximum((z[...] - mean) * scale, 0.0)
    out[...] = lax.dot_general(h, w[...], (((1,), (1,)), ((), ())),
                               preferred_element_type=jnp.float32) + b[...]


def _final(z, stats, w, b):
    return pl.pallas_call(
        _final_body,
        grid=(GRID,),
        in_specs=[
            pl.BlockSpec((BLK, D), lambda i: (i, 0)),
            pl.BlockSpec((2, D), lambda i: (0, 0)),
            pl.BlockSpec((NCLS, D), lambda i: (0, 0)),
            pl.BlockSpec((1, NCLS), lambda i: (0, 0)),
        ],
        out_specs=pl.BlockSpec((BLK, NCLS), lambda i: (i, 0)),
        out_shape=jax.ShapeDtypeStruct((N, NCLS), jnp.float32),
    )(z, stats, w, b)


def kernel(x, edge_index, W0, b0, W1, b1, W2, b2, W_last, b_last):
    dst = edge_index[0].astype(jnp.int32)
    src = edge_index[1].astype(jnp.int32)
    src2 = src.reshape(NW, EPN)
    dst2 = dst.reshape(NW, EPN)
    src_m = src2[:, :NCHUNK * K].reshape(NW, NCHUNK, K)
    dst_m = dst2[:, :NCHUNK * K].reshape(NW, NCHUNK, K)
    src_t = src2[:, NCHUNK * K:]
    dst_t = dst2[:, NCHUNK * K:]
    zrows = jnp.zeros((RPT, D), jnp.float32)

    h = x
    z = stats = None
    for li, (w, b) in enumerate(((W0, b0), (W1, b1), (W2, b2))):
        parts = _sc_spmm(h, src_m, dst_m, src_t, dst_t, zrows)
        z, stats = _linear_stats(parts[0], parts[1], w, b.reshape(1, D))
        if li < 2:
            h = _bn_relu(z, stats)
    return _final(z, stats, W_last, b_last.reshape(1, NCLS))


# speedup vs baseline: 3.5615x; 1.0400x over previous
import functools

import jax
import jax.numpy as jnp
from jax import lax
from jax.experimental import pallas as pl
from jax.experimental.pallas import tpu as pltpu
from jax.experimental.pallas import tpu_sc as plsc

N = 10000
E = 320000
D = 128
NCLS = 64
EPS_BN = 1e-5

NC = 2
NS = 16
NW = NC * NS
K = 128
EPN = E // NW
NCHUNK = 78
TE = EPN - NCHUNK * K
RPT = 624
TAIL0 = RPT * NS
TAIL = N - TAIL0
ACC_ROWS = N

BLK = 1000
GRID = N // BLK


def _sc_spmm_body(table, srcr, dstr, srct, dstt, zrows, out,
                  sidx, didx0, didx1, rows0, rows1, stail, dtail, rtail,
                  acc, g0, g1, d0, d1):
    c = lax.axis_index("c")
    s = lax.axis_index("s")
    wid = s * NC + c
    r0 = s * RPT

    @pl.when(c == 0)
    def _():
        pltpu.sync_copy(table.at[pl.ds(r0, RPT)], acc.at[pl.ds(r0, RPT)])

        @pl.when(s == NS - 1)
        def _():
            pltpu.sync_copy(table.at[pl.ds(TAIL0, TAIL)],
                            acc.at[pl.ds(TAIL0, TAIL)])

    @pl.when(c == 1)
    def _():
        pltpu.sync_copy(zrows.at[pl.ds(0, RPT)], acc.at[pl.ds(r0, RPT)])

        @pl.when(s == NS - 1)
        def _():
            pltpu.sync_copy(zrows.at[pl.ds(0, TAIL)],
                            acc.at[pl.ds(TAIL0, TAIL)])

    plsc.subcore_barrier()

    pltpu.sync_copy(srct.at[wid], stail)
    pltpu.sync_copy(dstt.at[wid], dtail)
    pltpu.async_copy(table.at[stail], rtail, g0).wait()
    pltpu.sync_copy(rtail, acc.at[dtail], add=True)

    pltpu.sync_copy(srcr.at[wid], sidx)

    def step(i, rows_c, g_c, didx_c, d_c, rows_n, g_n, didx_n, d_n):
        pltpu.make_async_copy(table.at[sidx.at[i]], rows_c, g_c).wait()

        @pl.when(i + 1 < NCHUNK)
        def _():
            pltpu.async_copy(dstr.at[wid].at[pl.ds(i + 1, 1)], didx_n, d_n)
            pltpu.async_copy(table.at[sidx.at[i + 1]], rows_n, g_n)

        pltpu.make_async_copy(
            dstr.at[wid].at[pl.ds(i, 1)], didx_c, d_c).wait()
        pltpu.sync_copy(rows_c, acc.at[didx_c.at[0]], add=True)

    pltpu.async_copy(dstr.at[wid].at[pl.ds(0, 1)], didx0, d0)
    pltpu.async_copy(table.at[sidx.at[0]], rows0, g0)

    def chunk(i, carry):
        @pl.when(i % 2 == 0)
        def _():
            step(i, rows0, g0, didx0, d0, rows1, g1, didx1, d1)

        @pl.when(i % 2 == 1)
        def _():
            step(i, rows1, g1, didx1, d1, rows0, g0, didx0, d0)

        return carry

    lax.fori_loop(0, NCHUNK, chunk, 0)

    plsc.subcore_barrier()
    pltpu.sync_copy(acc.at[pl.ds(r0, RPT)], out.at[c].at[pl.ds(r0, RPT)])

    @pl.when(s == NS - 1)
    def _():
        pltpu.sync_copy(acc.at[pl.ds(TAIL0, TAIL)],
                        out.at[c].at[pl.ds(TAIL0, TAIL)])


@functools.cache
def _sc_spmm_kernel():
    return pl.kernel(
        _sc_spmm_body,
        out_type=jax.ShapeDtypeStruct((NC, N, D), jnp.float32),
        mesh=plsc.VectorSubcoreMesh(
            core_axis_name="c", subcore_axis_name="s",
            num_cores=NC, num_subcores=NS),
        scratch_types=[
            pltpu.VMEM((NCHUNK, K), jnp.int32),
            pltpu.VMEM((1, K), jnp.int32),
            pltpu.VMEM((1, K), jnp.int32),
            pltpu.VMEM((K, D), jnp.float32),
            pltpu.VMEM((K, D), jnp.float32),
            pltpu.VMEM((TE,), jnp.int32),
            pltpu.VMEM((TE,), jnp.int32),
            pltpu.VMEM((TE, D), jnp.float32),
            pltpu.VMEM_SHARED((ACC_ROWS, D), jnp.float32),
            pltpu.SemaphoreType.DMA,
            pltpu.SemaphoreType.DMA,
            pltpu.SemaphoreType.DMA,
            pltpu.SemaphoreType.DMA,
        ],
    )


def _sc_spmm(h, src_m, dst_m, src_t, dst_t, zrows):
    return _sc_spmm_kernel()(h, src_m, dst_m, src_t, dst_t, zrows)


def _linear_stats_body(p, w, b, z, stats, s1, s2):
    i = pl.program_id(0)
    agg = p[0] + p[1]
    zz = lax.dot_general(agg, w[...], (((1,), (1,)), ((), ())),
                         preferred_element_type=jnp.float32) + b[...]
    z[...] = zz

    @pl.when(i == 0)
    def _():
        s1[...] = jnp.zeros_like(s1)
        s2[...] = jnp.zeros_like(s2)

    s1[...] += jnp.sum(zz, axis=0, keepdims=True)
    s2[...] += jnp.sum(zz * zz, axis=0, keepdims=True)

    @pl.when(i == pl.num_programs(0) - 1)
    def _():
        mean = s1[...] / N
        var = s2[...] / N - mean * mean
        stats[...] = jnp.concatenate(
            [mean, lax.rsqrt(var + EPS_BN)], axis=0)


def _linear_stats(p, w, b):
    return pl.pallas_call(
        _linear_stats_body,
        grid=(GRID,),
        in_specs=[
            pl.BlockSpec((NC, BLK, D), lambda i: (0, i, 0)),
            pl.BlockSpec((D, D), lambda i: (0, 0)),
            pl.BlockSpec((1, D), lambda i: (0, 0)),
        ],
        out_specs=[
            pl.BlockSpec((BLK, D), lambda i: (i, 0)),
            pl.BlockSpec((2, D), lambda i: (0, 0)),
        ],
        out_shape=[
            jax.ShapeDtypeStruct((N, D), jnp.float32),
            jax.ShapeDtypeStruct((2, D), jnp.float32),
        ],
        scratch_shapes=[
            pltpu.VMEM((1, D), jnp.float32),
            pltpu.VMEM((1, D), jnp.float32),
        ],
    )(p, w, b)


def _bn_relu_body(z, stats, h):
    mean = stats[0:1, :]
    scale = stats[1:2, :]
    h[...] = jnp.maximum((z[...] - mean) * scale, 0.0)


def _bn_relu(z, stats):
    return pl.pallas_call(
        _bn_relu_body,
        grid=(GRID,),
        in_specs=[
            pl.BlockSpec((BLK, D), lambda i: (i, 0)),
            pl.BlockSpec((2, D), lambda i: (0, 0)),
        ],
        out_specs=pl.BlockSpec((BLK, D), lambda i: (i, 0)),
        out_shape=jax.ShapeDtypeStruct((N, D), jnp.float32),
    )(z, stats)


def _final_body(z, stats, w, b, out):
    mean = stats[0:1, :]
    scale = stats[1:2, :]
    h = jnp.maximum((z[...] - mean) * scale, 0.0)
    out[...] = lax.dot_general(h, w[...], (((1,), (1,)), ((), ())),
                               preferred_element_type=jnp.float32) + b[...]


def _final(z, stats, w, b):
    return pl.pallas_call(
        _final_body,
        grid=(GRID,),
        in_specs=[
            pl.BlockSpec((BLK, D), lambda i: (i, 0)),
            pl.BlockSpec((2, D), lambda i: (0, 0)),
            pl.BlockSpec((NCLS, D), lambda i: (0, 0)),
            pl.BlockSpec((1, NCLS), lambda i: (0, 0)),
        ],
        out_specs=pl.BlockSpec((BLK, NCLS), lambda i: (i, 0)),
        out_shape=jax.ShapeDtypeStruct((N, NCLS), jnp.float32),
    )(z, stats, w, b)


def kernel(x, edge_index, W0, b0, W1, b1, W2, b2, W_last, b_last):
    dst = edge_index[0].astype(jnp.int32)
    src = edge_index[1].astype(jnp.int32)
    src2 = src.reshape(NW, EPN)
    dst2 = dst.reshape(NW, EPN)
    src_m = src2[:, :NCHUNK * K].reshape(NW, NCHUNK, K)
    dst_m = dst2[:, :NCHUNK * K].reshape(NW, NCHUNK, K)
    src_t = src2[:, NCHUNK * K:]
    dst_t = dst2[:, NCHUNK * K:]
    zrows = jnp.zeros((RPT, D), jnp.float32)

    h = x
    z = stats = None
    for li, (w, b) in enumerate(((W0, b0), (W1, b1), (W2, b2))):
        parts = _sc_spmm(h, src_m, dst_m, src_t, dst_t, zrows)
        z, stats = _linear_stats(parts, w, b.reshape(1, D))
        if li < 2:
            h = _bn_relu(z, stats)
    return _final(z, stats, W_last, b_last.reshape(1, NCLS))
